# baseline (speedup 1.0000x reference)
import functools

import jax
import jax.numpy as jnp
from jax import lax
from jax.experimental import pallas as pl
from jax.experimental.pallas import tpu as pltpu
from jax.experimental.pallas import tpu_sc as plsc

N = 10000
E = 160000
D_IN = 256
D_OUT = 256
K = 3

NC = 2
NS = 16
L = 16

DC = 64
NQ = 4
NP = 10240
EP = 163840
EBLK = 128
EBPT = EP // NS // EBLK
EHALF = EBPT // 2
NPT = NP // NS
NBLK = 128
NBPT = NPT // NBLK


def _rsqrt16(d):
    i = lax.bitcast_convert_type(d, jnp.int32)
    i = jnp.int32(0x5F3759DF) - (i >> 1)
    y = lax.bitcast_convert_type(i, jnp.float32)
    for _ in range(3):
        y = y * (jnp.float32(1.5) - jnp.float32(0.5) * d * y * y)
    return y


def _sc_body(x_hbm, src_hbm, dst_hbm, h_hbm, dis_hbm,
             g_sp, s_sp, src_v, dst_v, buf0, buf1, dis_sm,
             sem0, sem1, sem2, sem3):
    cid = lax.axis_index("c")
    sid = lax.axis_index("s")
    nbase = sid * NPT
    ebase = sid * EBPT

    def load_idx(h):
        pltpu.sync_copy(src_hbm.at[pl.ds(ebase + h * EHALF, EHALF)], src_v)
        pltpu.sync_copy(dst_hbm.at[pl.ds(ebase + h * EHALF, EHALF)], dst_v)

    def fill_ones(r, _):
        for c in range(DC // L):
            buf0[r, pl.ds(c * L, L)] = jnp.full((L,), 1.0, jnp.float32)
        return 0

    with jax.named_scope("ph_deg"):
        lax.fori_loop(0, NBLK, fill_ones, 0)

        def init_ones(b, _):
            pltpu.sync_copy(buf0, s_sp.at[pl.ds(nbase + b * NBLK, NBLK)])
            return 0

        lax.fori_loop(0, NBPT, init_ones, 0)
        plsc.subcore_barrier()

        def deg_start(j, b):
            pltpu.async_copy(buf0, s_sp.at[dst_v.at[j]], (sem2, sem3)[b],
                             add=True)

        def deg_wait(j, b):
            pltpu.make_async_copy(buf0, s_sp.at[dst_v.at[j]],
                                  (sem2, sem3)[b]).wait()

        def deg_pair(gp, _):
            for b in range(2):
                j = gp * 2 + b
                deg_start(j, b)

                @pl.when(j >= 1)
                def _():
                    deg_wait(j - 1, 1 - b)
            return 0

        for h in range(2):
            load_idx(h)
            lax.fori_loop(0, EHALF // 2, deg_pair, 0)
            deg_wait(EHALF - 1, (EHALF - 1) % 2)
        plsc.subcore_barrier()

    def make_dis(b, _):
        pltpu.sync_copy(s_sp.at[pl.ds(nbase + b * NBLK, NBLK)], buf0)

        def row(r, _):
            v = _rsqrt16(buf0[r, pl.ds(0, L)])
            dis_sm[b * NBLK + r] = v[0]
            buf0[r, pl.ds(0, L)] = v
            return 0

        lax.fori_loop(0, NBLK, row, 0)

        @pl.when(cid == 0)
        def _():
            pltpu.sync_copy(buf0, dis_hbm.at[pl.ds(nbase + b * NBLK, NBLK)])
        return 0

    with jax.named_scope("ph_dis"):
        lax.fori_loop(0, NBPT, make_dis, 0)

    def scale_rows(factor_fn, b):
        def row(r, _):
            f = factor_fn(dis_sm[b * NBLK + r])
            for c in range(DC // L):
                buf0[r, pl.ds(c * L, L)] = buf0[r, pl.ds(c * L, L)] * f
            return 0

        lax.fori_loop(0, NBLK, row, 0)

    bufs = (buf0, buf1)
    gsems = (sem0, sem1)
    ssems = (sem2, sem3)

    def gather_start(j, b):
        pltpu.async_copy(g_sp.at[src_v.at[j]], bufs[b], gsems[b])

    def gather_wait(j, b):
        pltpu.make_async_copy(g_sp.at[src_v.at[j]], bufs[b], gsems[b]).wait()

    def scatter_start(j, b):
        pltpu.async_copy(bufs[b], s_sp.at[dst_v.at[j]], ssems[b], add=True)

    def scatter_wait(j, b):
        pltpu.make_async_copy(bufs[b], s_sp.at[dst_v.at[j]], ssems[b]).wait()

    for p in range(2):
        qoff = (2 * cid + p) * NP

        def seed(b, _):
            rb = nbase + b * NBLK
            pltpu.sync_copy(x_hbm.at[pl.ds(qoff + rb, NBLK)], buf0)
            scale_rows(lambda s: s, b)
            pltpu.sync_copy(buf0, s_sp.at[pl.ds(rb, NBLK)])
            pltpu.sync_copy(buf0, g_sp.at[pl.ds(rb, NBLK)])
            return 0

        with jax.named_scope(f"ph_seed{p}"):
            lax.fori_loop(0, NBPT, seed, 0)
            plsc.subcore_barrier()

        for k in range(K):
            with jax.named_scope(f"ph_edges{p}{k}"):
                for h in range(2):
                    load_idx(h)
                    gather_start(0, 0)

                    def hop_pair(gp, _):
                        for b in range(2):
                            j = gp * 2 + b
                            gather_wait(j, b)
                            scatter_start(j, b)

                            @pl.when(j >= 1)
                            def _():
                                scatter_wait(j - 1, 1 - b)

                            @pl.when(j + 1 < EHALF)
                            def _():
                                gather_start(j + 1, 1 - b)
                        return 0

                    lax.fori_loop(0, EHALF // 2, hop_pair, 0)
                    scatter_wait(EHALF - 1, (EHALF - 1) % 2)
                plsc.subcore_barrier()

            last = k == K - 1

            def rescale(b, _):
                rb = nbase + b * NBLK
                pltpu.sync_copy(s_sp.at[pl.ds(rb, NBLK)], buf0)
                scale_rows(lambda s: s * s, b)
                pltpu.sync_copy(buf0, s_sp.at[pl.ds(rb, NBLK)])
                pltpu.sync_copy(buf0, g_sp.at[pl.ds(rb, NBLK)])
                return 0

            with jax.named_scope(f"ph_rescale{p}{k}"):
                if last:
                    pltpu.sync_copy(s_sp.at[pl.ds(nbase, NPT)],
                                    h_hbm.at[pl.ds(qoff + nbase, NPT)])
                else:
                    lax.fori_loop(0, NBPT, rescale, 0)
                    plsc.subcore_barrier()


@functools.partial(
    pl.kernel,
    out_type=(
        jax.ShapeDtypeStruct((NQ * NP, DC), jnp.float32),
        jax.ShapeDtypeStruct((NP, DC), jnp.float32),
    ),
    mesh=plsc.VectorSubcoreMesh(core_axis_name="c", subcore_axis_name="s"),
    scratch_types=[
        pltpu.VMEM_SHARED((NP, DC), jnp.float32),
        pltpu.VMEM_SHARED((NP, DC), jnp.float32),
        pltpu.VMEM((EHALF, EBLK), jnp.int32),
        pltpu.VMEM((EHALF, EBLK), jnp.int32),
        pltpu.VMEM((EBLK, DC), jnp.float32),
        pltpu.VMEM((EBLK, DC), jnp.float32),
        pltpu.SMEM((NPT,), jnp.float32),
        pltpu.SemaphoreType.DMA,
        pltpu.SemaphoreType.DMA,
        pltpu.SemaphoreType.DMA,
        pltpu.SemaphoreType.DMA,
    ],
)
def _sc_propagate(x_hbm, src_hbm, dst_hbm, h_hbm, dis_hbm,
                  g_sp, s_sp, src_v, dst_v, buf0, buf1, dis_sm,
                  sem0, sem1, sem2, sem3):
    _sc_body(x_hbm, src_hbm, dst_hbm, h_hbm, dis_hbm,
             g_sp, s_sp, src_v, dst_v, buf0, buf1, dis_sm,
             sem0, sem1, sem2, sem3)


RBLK = 1024


def _tc_body(h0_ref, h1_ref, h2_ref, h3_ref, d_ref, w_ref, b_ref, o_ref):
    d = d_ref[:, 0:1]
    acc = jnp.zeros((RBLK, D_OUT), jnp.float32) + b_ref[...]
    for q, href in enumerate((h0_ref, h1_ref, h2_ref, h3_ref)):
        acc += jnp.dot(href[...] * d, w_ref[q * DC:(q + 1) * DC, :],
                       preferred_element_type=jnp.float32)
    o_ref[...] = jax.nn.sigmoid(acc)


def kernel(x, edge_index, W, b):
    src = edge_index[0].astype(jnp.int32)
    dst = edge_index[1].astype(jnp.int32)
    srcp = jnp.pad(src, (0, EP - E), constant_values=NP - 1)
    dstp = jnp.pad(dst, (0, EP - E), constant_values=NP - 1)
    src2d = srcp.reshape(EP // EBLK, EBLK)
    dst2d = dstp.reshape(EP // EBLK, EBLK)
    xp = jnp.pad(x, ((0, NP - N), (0, 0)))
    x_flat = jnp.concatenate([xp[:, q * DC:(q + 1) * DC] for q in range(NQ)],
                             axis=0)

    h_flat, dis = _sc_propagate(x_flat, src2d, dst2d)

    nb = NP // RBLK
    out = pl.pallas_call(
        _tc_body,
        grid=(nb,),
        in_specs=[
            pl.BlockSpec((RBLK, DC), lambda i, q=q: (i + q * nb, 0))
            for q in range(NQ)
        ] + [
            pl.BlockSpec((RBLK, DC), lambda i: (i, 0)),
            pl.BlockSpec((D_IN, D_OUT), lambda i: (0, 0)),
            pl.BlockSpec((1, D_OUT), lambda i: (0, 0)),
        ],
        out_specs=pl.BlockSpec((RBLK, D_OUT), lambda i: (i, 0)),
        out_shape=jax.ShapeDtypeStruct((NP, D_OUT), jnp.float32),
    )(h_flat, h_flat, h_flat, h_flat, dis, W, b.reshape(1, D_OUT))
    return out[:N]


# speedup vs baseline: 1.0458x; 1.0458x over previous
import functools

import jax
import jax.numpy as jnp
from jax import lax
from jax.experimental import pallas as pl
from jax.experimental.pallas import tpu as pltpu
from jax.experimental.pallas import tpu_sc as plsc

N = 10000
E = 160000
D_IN = 256
D_OUT = 256
K = 3

NC = 2
NS = 16
L = 16

DC = 64
NQ = 4
NP = 10240
EP = 163840
EBLK = 128
EBPT = EP // NS // EBLK
EHALF = EBPT // 2
NPT = NP // NS
NBLK = 128
NBPT = NPT // NBLK


def _rsqrt16(d):
    i = lax.bitcast_convert_type(d, jnp.int32)
    i = jnp.int32(0x5F3759DF) - (i >> 1)
    y = lax.bitcast_convert_type(i, jnp.float32)
    for _ in range(3):
        y = y * (jnp.float32(1.5) - jnp.float32(0.5) * d * y * y)
    return y


def _sc_body(x_hbm, src_hbm, dst_hbm, h_hbm, dis_hbm,
             g_sp, s_sp, src_v, dst_v, buf0, buf1, dis_sm,
             sem0, sem1, sem2, sem3):
    cid = lax.axis_index("c")
    sid = lax.axis_index("s")
    nbase = sid * NPT
    ebase = sid * EBPT

    def load_idx(h):
        pltpu.sync_copy(src_hbm.at[pl.ds(ebase + h * EHALF, EHALF)], src_v)
        pltpu.sync_copy(dst_hbm.at[pl.ds(ebase + h * EHALF, EHALF)], dst_v)

    def fill_ones(r, _):
        for c in range(DC // L):
            buf0[r, pl.ds(c * L, L)] = jnp.full((L,), 1.0, jnp.float32)
        return 0

    with jax.named_scope("ph_deg"):
        lax.fori_loop(0, NBLK, fill_ones, 0)

        def init_ones(b, _):
            pltpu.sync_copy(buf0, s_sp.at[pl.ds(nbase + b * NBLK, NBLK)])
            return 0

        lax.fori_loop(0, NBPT, init_ones, 0)
        plsc.subcore_barrier()

        def deg_start(j, b):
            pltpu.async_copy(buf0, s_sp.at[dst_v.at[j]], (sem2, sem3)[b],
                             add=True)

        def deg_wait(j, b):
            pltpu.make_async_copy(buf0, s_sp.at[dst_v.at[j]],
                                  (sem2, sem3)[b]).wait()

        def deg_pair(gp, _):
            for b in range(2):
                j = gp * 2 + b
                deg_start(j, b)

                @pl.when(j >= 1)
                def _():
                    deg_wait(j - 1, 1 - b)
            return 0

        for h in range(2):
            load_idx(h)
            lax.fori_loop(0, EHALF // 2, deg_pair, 0)
            deg_wait(EHALF - 1, (EHALF - 1) % 2)
        plsc.subcore_barrier()

    def make_dis(b, _):
        pltpu.sync_copy(s_sp.at[pl.ds(nbase + b * NBLK, NBLK)], buf0)

        def row(r, _):
            v = _rsqrt16(buf0[r, pl.ds(0, L)])
            dis_sm[b * NBLK + r] = v[0]
            return 0

        lax.fori_loop(0, NBLK, row, 0)
        return 0

    with jax.named_scope("ph_dis"):
        @pl.when(cid == 0)
        def _():
            pltpu.sync_copy(s_sp.at[pl.ds(nbase, NPT)],
                            dis_hbm.at[pl.ds(nbase, NPT)])

        lax.fori_loop(0, NBPT, make_dis, 0)

    def scale_rows(buf, factor_fn, b):
        def row(r, _):
            f = factor_fn(dis_sm[b * NBLK + r])
            for c in range(DC // L):
                buf[r, pl.ds(c * L, L)] = buf[r, pl.ds(c * L, L)] * f
            return 0

        lax.fori_loop(0, NBLK, row, 0)

    bufs = (buf0, buf1)
    gsems = (sem0, sem1)
    ssems = (sem2, sem3)

    def staged_scale_pass(in_src_fn, out_dst_fns, factor_fn):
        def in_start(b):
            pltpu.async_copy(in_src_fn(b), bufs[b % 2], gsems[b % 2])

        def in_wait(b):
            pltpu.make_async_copy(in_src_fn(b), bufs[b % 2],
                                  gsems[b % 2]).wait()

        def outs_start(b):
            for f in out_dst_fns:
                pltpu.async_copy(bufs[b % 2], f(b), ssems[b % 2])

        def outs_wait(b):
            for f in out_dst_fns:
                pltpu.make_async_copy(bufs[b % 2], f(b), ssems[b % 2]).wait()

        in_start(0)
        for b in range(NBPT):
            in_wait(b)
            if b >= 1:
                outs_wait(b - 1)
            if b + 1 < NBPT:
                in_start(b + 1)
            scale_rows(bufs[b % 2], factor_fn, b)
            outs_start(b)
        outs_wait(NBPT - 1)

    def gather_start(j, b):
        pltpu.async_copy(g_sp.at[src_v.at[j]], bufs[b], gsems[b])

    def gather_wait(j, b):
        pltpu.make_async_copy(g_sp.at[src_v.at[j]], bufs[b], gsems[b]).wait()

    def scatter_start(j, b):
        pltpu.async_copy(bufs[b], s_sp.at[dst_v.at[j]], ssems[b], add=True)

    def scatter_wait(j, b):
        pltpu.make_async_copy(bufs[b], s_sp.at[dst_v.at[j]], ssems[b]).wait()

    for p in range(2):
        qoff = (2 * cid + p) * NP

        with jax.named_scope(f"ph_seed{p}"):
            staged_scale_pass(
                lambda b: x_hbm.at[pl.ds(qoff + nbase + b * NBLK, NBLK)],
                [lambda b: s_sp.at[pl.ds(nbase + b * NBLK, NBLK)],
                 lambda b: g_sp.at[pl.ds(nbase + b * NBLK, NBLK)]],
                lambda s: s,
            )
            plsc.subcore_barrier()

        for k in range(K):
            with jax.named_scope(f"ph_edges{p}{k}"):
                for h in range(2):
                    load_idx(h)
                    gather_start(0, 0)

                    def hop_pair(gp, _):
                        for b in range(2):
                            j = gp * 2 + b
                            gather_wait(j, b)
                            scatter_start(j, b)

                            @pl.when(j >= 1)
                            def _():
                                scatter_wait(j - 1, 1 - b)

                            @pl.when(j + 1 < EHALF)
                            def _():
                                gather_start(j + 1, 1 - b)
                        return 0

                    lax.fori_loop(0, EHALF // 2, hop_pair, 0)
                    scatter_wait(EHALF - 1, (EHALF - 1) % 2)
                plsc.subcore_barrier()

            last = k == K - 1

            with jax.named_scope(f"ph_rescale{p}{k}"):
                if last:
                    pltpu.sync_copy(s_sp.at[pl.ds(nbase, NPT)],
                                    h_hbm.at[pl.ds(qoff + nbase, NPT)])
                else:
                    staged_scale_pass(
                        lambda b: s_sp.at[pl.ds(nbase + b * NBLK, NBLK)],
                        [lambda b: s_sp.at[pl.ds(nbase + b * NBLK, NBLK)],
                         lambda b: g_sp.at[pl.ds(nbase + b * NBLK, NBLK)]],
                        lambda s: s * s,
                    )
                    plsc.subcore_barrier()


@functools.partial(
    pl.kernel,
    out_type=(
        jax.ShapeDtypeStruct((NQ * NP, DC), jnp.float32),
        jax.ShapeDtypeStruct((NP, DC), jnp.float32),
    ),
    mesh=plsc.VectorSubcoreMesh(core_axis_name="c", subcore_axis_name="s"),
    scratch_types=[
        pltpu.VMEM_SHARED((NP, DC), jnp.float32),
        pltpu.VMEM_SHARED((NP, DC), jnp.float32),
        pltpu.VMEM((EHALF, EBLK), jnp.int32),
        pltpu.VMEM((EHALF, EBLK), jnp.int32),
        pltpu.VMEM((EBLK, DC), jnp.float32),
        pltpu.VMEM((EBLK, DC), jnp.float32),
        pltpu.SMEM((NPT,), jnp.float32),
        pltpu.SemaphoreType.DMA,
        pltpu.SemaphoreType.DMA,
        pltpu.SemaphoreType.DMA,
        pltpu.SemaphoreType.DMA,
    ],
)
def _sc_propagate(x_hbm, src_hbm, dst_hbm, h_hbm, dis_hbm,
                  g_sp, s_sp, src_v, dst_v, buf0, buf1, dis_sm,
                  sem0, sem1, sem2, sem3):
    _sc_body(x_hbm, src_hbm, dst_hbm, h_hbm, dis_hbm,
             g_sp, s_sp, src_v, dst_v, buf0, buf1, dis_sm,
             sem0, sem1, sem2, sem3)


RBLK = 1024


def _tc_body(h0_ref, h1_ref, h2_ref, h3_ref, d_ref, w_ref, b_ref, o_ref):
    d = lax.rsqrt(d_ref[:, 0:1])
    acc = jnp.zeros((RBLK, D_OUT), jnp.float32) + b_ref[...]
    for q, href in enumerate((h0_ref, h1_ref, h2_ref, h3_ref)):
        acc += jnp.dot(href[...] * d, w_ref[q * DC:(q + 1) * DC, :],
                       preferred_element_type=jnp.float32)
    o_ref[...] = jax.nn.sigmoid(acc)


def kernel(x, edge_index, W, b):
    src = edge_index[0].astype(jnp.int32)
    dst = edge_index[1].astype(jnp.int32)
    srcp = jnp.pad(src, (0, EP - E), constant_values=NP - 1)
    dstp = jnp.pad(dst, (0, EP - E), constant_values=NP - 1)
    src2d = srcp.reshape(EP // EBLK, EBLK)
    dst2d = dstp.reshape(EP // EBLK, EBLK)
    xp = jnp.pad(x, ((0, NP - N), (0, 0)))
    x_flat = jnp.concatenate([xp[:, q * DC:(q + 1) * DC] for q in range(NQ)],
                             axis=0)

    h_flat, dis = _sc_propagate(x_flat, src2d, dst2d)

    nb = NP // RBLK
    out = pl.pallas_call(
        _tc_body,
        grid=(nb,),
        in_specs=[
            pl.BlockSpec((RBLK, DC), lambda i, q=q: (i + q * nb, 0))
            for q in range(NQ)
        ] + [
            pl.BlockSpec((RBLK, DC), lambda i: (i, 0)),
            pl.BlockSpec((D_IN, D_OUT), lambda i: (0, 0)),
            pl.BlockSpec((1, D_OUT), lambda i: (0, 0)),
        ],
        out_specs=pl.BlockSpec((RBLK, D_OUT), lambda i: (i, 0)),
        out_shape=jax.ShapeDtypeStruct((NP, D_OUT), jnp.float32),
    )(h_flat, h_flat, h_flat, h_flat, dis, W, b.reshape(1, D_OUT))
    return out[:N]
